# Initial kernel scaffold; baseline (speedup 1.0000x reference)
import functools

import jax
import jax.numpy as jnp
from jax.experimental import pallas as pl
from jax.experimental.pallas import tpu as pltpu

B, L, D = 1, 2048, 1024
F = 4096
E = 8
K = 2

ROW_TILE = 256
F_TILE = 2048


def _router_body(x_ref, rn_g_ref, rn_b_ref, wr_ref, br_ref, xn_ref, c_ref):
    x = x_ref[...]
    mu = jnp.mean(x, axis=-1, keepdims=True)
    xc = x - mu
    var = jnp.mean(xc * xc, axis=-1, keepdims=True)
    xn = xc * jax.lax.rsqrt(var + 1e-5)
    xn_ref[...] = xn
    xa = xn * rn_g_ref[...] + rn_b_ref[...]
    logits = jax.lax.dot_general(
        xa, wr_ref[...], (((1,), (0,)), ((), ())),
        preferred_element_type=jnp.float32) + br_ref[...]
    lane = jax.lax.broadcasted_iota(jnp.int32, logits.shape, 1)
    m1 = jnp.max(logits, axis=-1, keepdims=True)
    i1 = jnp.min(jnp.where(logits == m1, lane, E), axis=-1, keepdims=True)
    l2 = jnp.where(lane == i1, -jnp.inf, logits)
    m2 = jnp.max(l2, axis=-1, keepdims=True)
    i2 = jnp.min(jnp.where(l2 == m2, lane, E), axis=-1, keepdims=True)
    w1 = 1.0 / (1.0 + jnp.exp(m2 - m1))
    w2 = 1.0 - w1
    c_ref[...] = jnp.where(lane == i1, w1, 0.0) + jnp.where(lane == i2, w2, 0.0)


def _router(flat, rn_g, rn_b, Wr, br):
    grid = (L // ROW_TILE,)
    return pl.pallas_call(
        _router_body,
        grid=grid,
        in_specs=[
            pl.BlockSpec((ROW_TILE, D), lambda i: (i, 0)),
            pl.BlockSpec((1, D), lambda i: (0, 0)),
            pl.BlockSpec((1, D), lambda i: (0, 0)),
            pl.BlockSpec((D, E), lambda i: (0, 0)),
            pl.BlockSpec((1, E), lambda i: (0, 0)),
        ],
        out_specs=[
            pl.BlockSpec((ROW_TILE, D), lambda i: (i, 0)),
            pl.BlockSpec((ROW_TILE, E), lambda i: (i, 0)),
        ],
        out_shape=[
            jax.ShapeDtypeStruct((L, D), jnp.float32),
            jax.ShapeDtypeStruct((L, E), jnp.float32),
        ],
    )(flat, rn_g.reshape(1, D), rn_b.reshape(1, D), Wr, br.reshape(1, E))


def _gelu(x):
    return 0.5 * x * (1.0 + jax.lax.erf(x * 0.7071067811865476))


def _ffn_body(xn_ref, c_ref, lng_ref, lnb_ref, w1_ref, b1_ref, w2_ref,
              b2_ref, out_ref):
    e_i = pl.program_id(0)
    f_i = pl.program_id(1)

    @pl.when(jnp.logical_and(e_i == 0, f_i == 0))
    def _():
        out_ref[...] = jnp.zeros_like(out_ref)

    xa = xn_ref[...] * lng_ref[0] + lnb_ref[0]
    h = jax.lax.dot_general(
        xa, w1_ref[0], (((1,), (0,)), ((), ())),
        preferred_element_type=jnp.float32) + b1_ref[0]
    h = _gelu(h)
    o = jax.lax.dot_general(
        h, w2_ref[0], (((1,), (0,)), ((), ())),
        preferred_element_type=jnp.float32)

    @pl.when(f_i == 0)
    def _():
        out_ref[...] += b2_ref[0]

    lane = jax.lax.broadcasted_iota(jnp.int32, (L, E), 1)
    w = jnp.sum(jnp.where(lane == e_i, c_ref[...], 0.0), axis=-1,
                keepdims=True)
    out_ref[...] += w * o


def _ffn(xn, c, ln_g, ln_b, W1, b1, W2, b2):
    nf = F // F_TILE
    grid = (E, nf)
    return pl.pallas_call(
        _ffn_body,
        grid=grid,
        in_specs=[
            pl.BlockSpec((L, D), lambda e, f: (0, 0)),
            pl.BlockSpec((L, E), lambda e, f: (0, 0)),
            pl.BlockSpec((1, D), lambda e, f: (e, 0)),
            pl.BlockSpec((1, D), lambda e, f: (e, 0)),
            pl.BlockSpec((1, D, F_TILE), lambda e, f: (e, 0, f)),
            pl.BlockSpec((1, F_TILE), lambda e, f: (e, f)),
            pl.BlockSpec((1, F_TILE, D), lambda e, f: (e, f, 0)),
            pl.BlockSpec((1, D), lambda e, f: (e, 0)),
        ],
        out_specs=pl.BlockSpec((L, D), lambda e, f: (0, 0)),
        out_shape=jax.ShapeDtypeStruct((L, D), jnp.float32),
    )(xn, c, ln_g, ln_b, W1, b1, W2, b2)


def kernel(hidden_states, rn_g, rn_b, Wr, br, ln_g, ln_b, W1, b1, W2, b2):
    flat = hidden_states.reshape(L, D)
    xn, c = _router(flat, rn_g, rn_b, Wr, br)
    out = _ffn(xn, c, ln_g, ln_b, W1, b1, W2, b2)
    return out.reshape(B, L, D)


# speedup vs baseline: 2.7650x; 2.7650x over previous
import functools

import jax
import jax.numpy as jnp
from jax.experimental import pallas as pl
from jax.experimental.pallas import tpu as pltpu

B, L, D = 1, 2048, 1024
F = 4096
E = 8
K = 2

ROW_TILE = 256
F_TILE = 512


def _router_body(x_ref, rn_g_ref, rn_b_ref, wr_ref, br_ref, xn_ref, c_ref):
    x = x_ref[...]
    mu = jnp.mean(x, axis=-1, keepdims=True)
    xc = x - mu
    var = jnp.mean(xc * xc, axis=-1, keepdims=True)
    xn = xc * jax.lax.rsqrt(var + 1e-5)
    xn_ref[...] = xn
    xa = xn * rn_g_ref[...] + rn_b_ref[...]
    logits = jax.lax.dot_general(
        xa, wr_ref[...], (((1,), (0,)), ((), ())),
        preferred_element_type=jnp.float32) + br_ref[...]
    lane = jax.lax.broadcasted_iota(jnp.int32, logits.shape, 1)
    m1 = jnp.max(logits, axis=-1, keepdims=True)
    i1 = jnp.min(jnp.where(logits == m1, lane, E), axis=-1, keepdims=True)
    l2 = jnp.where(lane == i1, -jnp.inf, logits)
    m2 = jnp.max(l2, axis=-1, keepdims=True)
    i2 = jnp.min(jnp.where(l2 == m2, lane, E), axis=-1, keepdims=True)
    w1 = 1.0 / (1.0 + jnp.exp(m2 - m1))
    w2 = 1.0 - w1
    c_ref[...] = jnp.where(lane == i1, w1, 0.0) + jnp.where(lane == i2, w2, 0.0)


def _router(flat, rn_g, rn_b, Wr, br):
    grid = (L // ROW_TILE,)
    return pl.pallas_call(
        _router_body,
        grid=grid,
        in_specs=[
            pl.BlockSpec((ROW_TILE, D), lambda i: (i, 0)),
            pl.BlockSpec((1, D), lambda i: (0, 0)),
            pl.BlockSpec((1, D), lambda i: (0, 0)),
            pl.BlockSpec((D, E), lambda i: (0, 0)),
            pl.BlockSpec((1, E), lambda i: (0, 0)),
        ],
        out_specs=[
            pl.BlockSpec((ROW_TILE, D), lambda i: (i, 0)),
            pl.BlockSpec((ROW_TILE, E), lambda i: (i, 0)),
        ],
        out_shape=[
            jax.ShapeDtypeStruct((L, D), jnp.float32),
            jax.ShapeDtypeStruct((L, E), jnp.float32),
        ],
    )(flat, rn_g.reshape(1, D), rn_b.reshape(1, D), Wr, br.reshape(1, E))


def _gelu(x):
    return 0.5 * x * (1.0 + jax.lax.erf(x * 0.7071067811865476))


def _ffn_body(xn_ref, c_ref, lng_ref, lnb_ref, w1_ref, b1_ref, w2_ref,
              b2_ref, out_ref):
    e_i = pl.program_id(0)
    f_i = pl.program_id(1)

    @pl.when(jnp.logical_and(e_i == 0, f_i == 0))
    def _():
        out_ref[...] = jnp.zeros_like(out_ref)

    xa = xn_ref[...] * lng_ref[0] + lnb_ref[0]
    h = jax.lax.dot_general(
        xa, w1_ref[0], (((1,), (0,)), ((), ())),
        preferred_element_type=jnp.float32) + b1_ref[0]
    h = _gelu(h)
    o = jax.lax.dot_general(
        h, w2_ref[0], (((1,), (0,)), ((), ())),
        preferred_element_type=jnp.float32)
    o = o + jnp.where(f_i == 0, b2_ref[0], 0.0)

    lane = jax.lax.broadcasted_iota(jnp.int32, (L, E), 1)
    w = jnp.sum(jnp.where(lane == e_i, c_ref[...], 0.0), axis=-1,
                keepdims=True)
    out_ref[...] += w * o


def _ffn(xn, c, ln_g, ln_b, W1, b1, W2, b2):
    nf = F // F_TILE
    grid = (E, nf)
    return pl.pallas_call(
        _ffn_body,
        grid=grid,
        in_specs=[
            pl.BlockSpec((L, D), lambda e, f: (0, 0)),
            pl.BlockSpec((L, E), lambda e, f: (0, 0)),
            pl.BlockSpec((1, 1, D), lambda e, f: (e, 0, 0)),
            pl.BlockSpec((1, 1, D), lambda e, f: (e, 0, 0)),
            pl.BlockSpec((1, D, F_TILE), lambda e, f: (e, 0, f)),
            pl.BlockSpec((1, 1, F_TILE), lambda e, f: (e, 0, f)),
            pl.BlockSpec((1, F_TILE, D), lambda e, f: (e, f, 0)),
            pl.BlockSpec((1, 1, D), lambda e, f: (e, 0, 0)),
        ],
        out_specs=pl.BlockSpec((L, D), lambda e, f: (0, 0)),
        out_shape=jax.ShapeDtypeStruct((L, D), jnp.float32),
    )(xn, c, ln_g.reshape(E, 1, D), ln_b.reshape(E, 1, D), W1,
      b1.reshape(E, 1, F), W2, b2.reshape(E, 1, D))


def kernel(hidden_states, rn_g, rn_b, Wr, br, ln_g, ln_b, W1, b1, W2, b2):
    flat = hidden_states.reshape(L, D)
    xn, c = _router(flat, rn_g, rn_b, Wr, br)
    out = _ffn(xn, c, ln_g, ln_b, W1, b1, W2, b2)
    return out.reshape(B, L, D)


# speedup vs baseline: 2.8559x; 1.0329x over previous
import functools

import jax
import jax.numpy as jnp
from jax.experimental import pallas as pl
from jax.experimental.pallas import tpu as pltpu

B, L, D = 1, 2048, 1024
F = 4096
E = 8
K = 2

ROW_TILE = 256
F_TILE = 512


def _router_body(x_ref, rn_g_ref, rn_b_ref, wr_ref, br_ref, xn_ref, c_ref):
    x = x_ref[...]
    mu = jnp.mean(x, axis=-1, keepdims=True)
    xc = x - mu
    var = jnp.mean(xc * xc, axis=-1, keepdims=True)
    xn = xc * jax.lax.rsqrt(var + 1e-5)
    xn_ref[...] = xn
    xa = xn * rn_g_ref[...] + rn_b_ref[...]
    logits = jax.lax.dot_general(
        xa, wr_ref[...], (((1,), (0,)), ((), ())),
        preferred_element_type=jnp.float32) + br_ref[...]
    lane = jax.lax.broadcasted_iota(jnp.int32, logits.shape, 1)
    m1 = jnp.max(logits, axis=-1, keepdims=True)
    i1 = jnp.min(jnp.where(logits == m1, lane, E), axis=-1, keepdims=True)
    l2 = jnp.where(lane == i1, -jnp.inf, logits)
    m2 = jnp.max(l2, axis=-1, keepdims=True)
    i2 = jnp.min(jnp.where(l2 == m2, lane, E), axis=-1, keepdims=True)
    w1 = 1.0 / (1.0 + jnp.exp(m2 - m1))
    w2 = 1.0 - w1
    c_ref[...] = jnp.where(lane == i1, w1, 0.0) + jnp.where(lane == i2, w2, 0.0)


def _router(flat, rn_g, rn_b, Wr, br):
    grid = (L // ROW_TILE,)
    return pl.pallas_call(
        _router_body,
        grid=grid,
        in_specs=[
            pl.BlockSpec((ROW_TILE, D), lambda i: (i, 0)),
            pl.BlockSpec((1, D), lambda i: (0, 0)),
            pl.BlockSpec((1, D), lambda i: (0, 0)),
            pl.BlockSpec((D, E), lambda i: (0, 0)),
            pl.BlockSpec((1, E), lambda i: (0, 0)),
        ],
        out_specs=[
            pl.BlockSpec((ROW_TILE, D), lambda i: (i, 0)),
            pl.BlockSpec((ROW_TILE, E), lambda i: (i, 0)),
        ],
        out_shape=[
            jax.ShapeDtypeStruct((L, D), jnp.float32),
            jax.ShapeDtypeStruct((L, E), jnp.float32),
        ],
    )(flat, rn_g.reshape(1, D), rn_b.reshape(1, D), Wr, br.reshape(1, E))


def _gelu(x):
    return 0.5 * x * (1.0 + jax.lax.erf(x * 0.7071067811865476))


def _ffn_body(xn_ref, c_ref, lng_ref, lnb_ref, w1_ref, b1_ref, w2_ref,
              b2_ref, out_ref):
    e_i = pl.program_id(0)
    f_i = pl.program_id(1)

    @pl.when(jnp.logical_and(e_i == 0, f_i == 0))
    def _():
        out_ref[...] = jnp.zeros_like(out_ref)

    xa = xn_ref[...] * lng_ref[0] + lnb_ref[0]
    h = jax.lax.dot_general(
        xa.astype(jnp.bfloat16), w1_ref[0].astype(jnp.bfloat16),
        (((1,), (0,)), ((), ())),
        preferred_element_type=jnp.float32) + b1_ref[0]
    h = _gelu(h)
    o = jax.lax.dot_general(
        h.astype(jnp.bfloat16), w2_ref[0].astype(jnp.bfloat16),
        (((1,), (0,)), ((), ())),
        preferred_element_type=jnp.float32)
    o = o + jnp.where(f_i == 0, b2_ref[0], 0.0)

    lane = jax.lax.broadcasted_iota(jnp.int32, (L, E), 1)
    w = jnp.sum(jnp.where(lane == e_i, c_ref[...], 0.0), axis=-1,
                keepdims=True)
    out_ref[...] += w * o


def _ffn(xn, c, ln_g, ln_b, W1, b1, W2, b2):
    nf = F // F_TILE
    grid = (E, nf)
    return pl.pallas_call(
        _ffn_body,
        grid=grid,
        in_specs=[
            pl.BlockSpec((L, D), lambda e, f: (0, 0)),
            pl.BlockSpec((L, E), lambda e, f: (0, 0)),
            pl.BlockSpec((1, 1, D), lambda e, f: (e, 0, 0)),
            pl.BlockSpec((1, 1, D), lambda e, f: (e, 0, 0)),
            pl.BlockSpec((1, D, F_TILE), lambda e, f: (e, 0, f)),
            pl.BlockSpec((1, 1, F_TILE), lambda e, f: (e, 0, f)),
            pl.BlockSpec((1, F_TILE, D), lambda e, f: (e, f, 0)),
            pl.BlockSpec((1, 1, D), lambda e, f: (e, 0, 0)),
        ],
        out_specs=pl.BlockSpec((L, D), lambda e, f: (0, 0)),
        out_shape=jax.ShapeDtypeStruct((L, D), jnp.float32),
    )(xn, c, ln_g.reshape(E, 1, D), ln_b.reshape(E, 1, D), W1,
      b1.reshape(E, 1, F), W2, b2.reshape(E, 1, D))


def kernel(hidden_states, rn_g, rn_b, Wr, br, ln_g, ln_b, W1, b1, W2, b2):
    flat = hidden_states.reshape(L, D)
    xn, c = _router(flat, rn_g, rn_b, Wr, br)
    out = _ffn(xn, c, ln_g, ln_b, W1, b1, W2, b2)
    return out.reshape(B, L, D)


# speedup vs baseline: 3.2290x; 1.1306x over previous
import functools

import jax
import jax.numpy as jnp
from jax import lax
from jax.experimental import pallas as pl
from jax.experimental.pallas import tpu as pltpu
from jax.experimental.pallas import tpu_sc as plsc

B, L, D = 1, 2048, 1024
F = 4096
E = 8
K = 2

ROW_TILE = 256
TM = 128
NT = (K * L) // TM + E
RPAD = NT * TM
F_TILE = 2048
NF = F // F_TILE
NSC = 32
A = K * L


def _router_body(x_ref, rn_g_ref, rn_b_ref, wr_ref, br_ref,
                 xn_ref, i1_ref, i2_ref, w1_ref, w2_ref):
    x = x_ref[...]
    mu = jnp.mean(x, axis=-1, keepdims=True)
    xc = x - mu
    var = jnp.mean(xc * xc, axis=-1, keepdims=True)
    xn = xc * lax.rsqrt(var + 1e-5)
    xn_ref[...] = xn
    xa = xn * rn_g_ref[...] + rn_b_ref[...]
    logits = lax.dot_general(
        xa, wr_ref[...], (((1,), (0,)), ((), ())),
        preferred_element_type=jnp.float32) + br_ref[...]
    lane = lax.broadcasted_iota(jnp.int32, logits.shape, 1)
    m1 = jnp.max(logits, axis=-1, keepdims=True)
    i1 = jnp.min(jnp.where(logits == m1, lane, E), axis=-1, keepdims=True)
    l2 = jnp.where(lane == i1, -jnp.inf, logits)
    m2 = jnp.max(l2, axis=-1, keepdims=True)
    i2 = jnp.min(jnp.where(l2 == m2, lane, E), axis=-1, keepdims=True)
    w1 = 1.0 / (1.0 + jnp.exp(m2 - m1))
    w2 = 1.0 - w1
    i1_ref[...] = i1
    i2_ref[...] = i2
    ones = jnp.ones((1, 128), jnp.float32)
    w1_ref[...] = w1 * ones
    w2_ref[...] = w2 * ones


def _router(flat, rn_g, rn_b, Wr, br):
    return pl.pallas_call(
        _router_body,
        grid=(L // ROW_TILE,),
        in_specs=[
            pl.BlockSpec((ROW_TILE, D), lambda i: (i, 0)),
            pl.BlockSpec((1, D), lambda i: (0, 0)),
            pl.BlockSpec((1, D), lambda i: (0, 0)),
            pl.BlockSpec((D, E), lambda i: (0, 0)),
            pl.BlockSpec((1, E), lambda i: (0, 0)),
        ],
        out_specs=[
            pl.BlockSpec((ROW_TILE, D), lambda i: (i, 0)),
            pl.BlockSpec((ROW_TILE, 1), lambda i: (i, 0)),
            pl.BlockSpec((ROW_TILE, 1), lambda i: (i, 0)),
            pl.BlockSpec((ROW_TILE, 128), lambda i: (i, 0)),
            pl.BlockSpec((ROW_TILE, 128), lambda i: (i, 0)),
        ],
        out_shape=[
            jax.ShapeDtypeStruct((L, D), jnp.float32),
            jax.ShapeDtypeStruct((L, 1), jnp.int32),
            jax.ShapeDtypeStruct((L, 1), jnp.int32),
            jax.ShapeDtypeStruct((L, 128), jnp.float32),
            jax.ShapeDtypeStruct((L, 128), jnp.float32),
        ],
    )(flat, rn_g.reshape(1, D), rn_b.reshape(1, D), Wr, br.reshape(1, E))


def _meta_body(e1_ref, e2_ref, pos_ref, te_ref):
    eid = jnp.concatenate([e1_ref[...], e2_ref[...]], axis=0)
    r_iota = lax.broadcasted_iota(jnp.int32, (128, 128), 0)
    c_iota = lax.broadcasted_iota(jnp.int32, (128, 128), 1)
    upper = (r_iota < c_iota).astype(jnp.float32)
    r2 = lax.broadcasted_iota(jnp.int32, (32, 32), 0)
    c2 = lax.broadcasted_iota(jnp.int32, (32, 32), 1)
    lower = (c2 < r2).astype(jnp.float32)

    rank = jnp.zeros((32, 128), jnp.float32)
    offsel = jnp.zeros((32, 128), jnp.float32)
    run = jnp.float32(0.0)
    ends = []
    for e in range(E):
        m = (eid == e).astype(jnp.float32)
        pre = lax.dot_general(m, upper, (((1,), (0,)), ((), ())),
                              preferred_element_type=jnp.float32)
        s = jnp.sum(m, axis=1, keepdims=True)
        roff = lax.dot_general(lower, s, (((1,), (0,)), ((), ())),
                               preferred_element_type=jnp.float32)
        rank = rank + m * (pre + roff)
        offsel = offsel + m * run
        c = jnp.sum(m)
        cpad = jnp.ceil(c / TM) * TM
        run = run + cpad
        ends.append(run)
    pos_ref[...] = (rank + offsel).astype(jnp.int32)

    iv = (lax.broadcasted_iota(jnp.int32, (1, NT), 1) * TM).astype(
        jnp.float32)
    te = jnp.zeros((1, NT), jnp.float32)
    for e in range(E - 1):
        te = te + (iv >= ends[e]).astype(jnp.float32)
    te_ref[...] = te.astype(jnp.int32)


def _metadata(e1r, e2r):
    return pl.pallas_call(
        _meta_body,
        out_shape=[
            jax.ShapeDtypeStruct((32, 128), jnp.int32),
            jax.ShapeDtypeStruct((1, NT), jnp.int32),
        ],
    )(e1r, e2r)


def _sc_scatter_body(xn_hbm, pos_hbm, wb_hbm, xg_hbm, wg_hbm,
                     idx_v, rows_v, wrow_v, sem):
    wid = lax.axis_index("s") * 2 + lax.axis_index("c")
    pltpu.sync_copy(pos_hbm.at[wid], idx_v)
    src0 = lax.rem(wid, 16) * 128
    for j in range(4):
        pltpu.sync_copy(xn_hbm.at[pl.ds(src0 + j * 32, 32)], rows_v)
        pltpu.async_copy(rows_v, xg_hbm.at[idx_v.at[j]], sem).wait()
        pltpu.sync_copy(wb_hbm.at[wid, j], wrow_v)
        pltpu.async_copy(wrow_v, wg_hbm.at[idx_v.at[j]], sem).wait()


def _sc_scatter(xn, pos3, wb):
    mesh = plsc.VectorSubcoreMesh(core_axis_name="c", subcore_axis_name="s")
    fn = functools.partial(
        pl.kernel,
        mesh=mesh,
        out_type=[
            jax.ShapeDtypeStruct((RPAD, D), jnp.float32),
            jax.ShapeDtypeStruct((RPAD, 128), jnp.float32),
        ],
        scratch_types=[
            pltpu.VMEM((4, 32), jnp.int32),
            pltpu.VMEM((32, D), jnp.float32),
            pltpu.VMEM((32, 128), jnp.float32),
            pltpu.SemaphoreType.DMA,
        ],
    )(_sc_scatter_body)
    return fn(xn, pos3, wb)


def _sc_combine_body(og_hbm, p1_hbm, p2_hbm, p1h_hbm, p2h_hbm, out_hbm,
                     idx_v, acc_v, tmp_v, sem):
    wid = lax.axis_index("s") * 2 + lax.axis_index("c")
    pltpu.sync_copy(p1_hbm.at[wid], idx_v.at[0])
    pltpu.sync_copy(p2_hbm.at[wid], idx_v.at[1])
    pltpu.sync_copy(p1h_hbm.at[wid], idx_v.at[2])
    pltpu.sync_copy(p2h_hbm.at[wid], idx_v.at[3])

    def accum(cc, carry):
        off = cc * 16
        for i in range(32):
            acc_v[i, pl.ds(off, 16)] = (
                acc_v[i, pl.ds(off, 16)] + tmp_v[i, pl.ds(off, 16)])
        return carry

    for j in range(2):
        pltpu.async_copy(og_hbm.at[idx_v.at[0, j]], acc_v, sem).wait()
        for q in range(1, 4):
            pltpu.async_copy(og_hbm.at[idx_v.at[q, j]], tmp_v, sem).wait()
            lax.fori_loop(0, D // 16, accum, 0)
        pltpu.sync_copy(acc_v, out_hbm.at[pl.ds(wid * 64 + j * 32, 32)])


def _sc_combine(og2, p1g, p2g, p1h, p2h):
    mesh = plsc.VectorSubcoreMesh(core_axis_name="c", subcore_axis_name="s")
    fn = functools.partial(
        pl.kernel,
        mesh=mesh,
        out_type=jax.ShapeDtypeStruct((L, D), jnp.float32),
        scratch_types=[
            pltpu.VMEM((4, 2, 32), jnp.int32),
            pltpu.VMEM((32, D), jnp.float32),
            pltpu.VMEM((32, D), jnp.float32),
            pltpu.SemaphoreType.DMA,
        ],
    )(_sc_combine_body)
    return fn(og2, p1g, p2g, p1h, p2h)


def _gelu(x):
    return 0.5 * x * (1.0 + lax.erf(x * 0.7071067811865476))


def _ffn_body(te_ref, xg_ref, wg_ref, lng_ref, lnb_ref, w1_ref, b1_ref,
              w2_ref, b2_ref, out_ref):
    f = pl.program_id(0)
    xa = xg_ref[...] * lng_ref[0] + lnb_ref[0]
    h = lax.dot_general(
        xa, w1_ref[0], (((1,), (0,)), ((), ())),
        preferred_element_type=jnp.float32) + b1_ref[0]
    h = _gelu(h)
    o = lax.dot_general(
        h, w2_ref[0], (((1,), (0,)), ((), ())),
        preferred_element_type=jnp.float32)
    o = o + jnp.where(f == 0, b2_ref[0], jnp.zeros_like(b2_ref[0]))
    out_ref[0] = wg_ref[...][:, 0:1] * o


def _grouped_ffn(te1d, xg, wg, ln_g, ln_b, W1, b1, W2, b2):
    grid_spec = pltpu.PrefetchScalarGridSpec(
        num_scalar_prefetch=1,
        grid=(NF, NT),
        in_specs=[
            pl.BlockSpec((TM, D), lambda f, j, te: (j, 0)),
            pl.BlockSpec((TM, 128), lambda f, j, te: (j, 0)),
            pl.BlockSpec((1, 1, D), lambda f, j, te: (te[j], 0, 0)),
            pl.BlockSpec((1, 1, D), lambda f, j, te: (te[j], 0, 0)),
            pl.BlockSpec((1, D, F_TILE), lambda f, j, te: (te[j], 0, f)),
            pl.BlockSpec((1, 1, F_TILE), lambda f, j, te: (te[j], 0, f)),
            pl.BlockSpec((1, F_TILE, D), lambda f, j, te: (te[j], f, 0)),
            pl.BlockSpec((1, 1, D), lambda f, j, te: (te[j], 0, 0)),
        ],
        out_specs=pl.BlockSpec((1, TM, D), lambda f, j, te: (f, j, 0)),
    )
    return pl.pallas_call(
        _ffn_body,
        grid_spec=grid_spec,
        out_shape=jax.ShapeDtypeStruct((NF, RPAD, D), jnp.float32),
    )(te1d, xg, wg, ln_g.reshape(E, 1, D), ln_b.reshape(E, 1, D),
      W1, b1.reshape(E, 1, F), W2, b2.reshape(E, 1, D))


def kernel(hidden_states, rn_g, rn_b, Wr, br, ln_g, ln_b, W1, b1, W2, b2):
    flat = hidden_states.reshape(L, D)
    xn, i1, i2, w1b, w2b = _router(flat, rn_g, rn_b, Wr, br)

    pos2d, te = _metadata(i1.reshape(16, 128), i2.reshape(16, 128))
    pos3 = pos2d.reshape(NSC, 4, 32)
    p1g = pos2d[:16].reshape(NSC, 2, 32)
    p2g = pos2d[16:].reshape(NSC, 2, 32)
    wb = jnp.concatenate([w1b, w2b], axis=0).reshape(NSC, 4, 32, 128)

    xg, wg = _sc_scatter(xn, pos3, wb)
    og = _grouped_ffn(te.reshape(NT), xg, wg, ln_g, ln_b, W1, b1, W2, b2)
    final = _sc_combine(og.reshape(NF * RPAD, D), p1g, p2g,
                        p1g + RPAD, p2g + RPAD)
    return final.reshape(B, L, D)


# speedup vs baseline: 3.3132x; 1.0261x over previous
import functools

import jax
import jax.numpy as jnp
from jax import lax
from jax.experimental import pallas as pl
from jax.experimental.pallas import tpu as pltpu
from jax.experimental.pallas import tpu_sc as plsc

B, L, D = 1, 2048, 1024
F = 4096
E = 8
K = 2

ROW_TILE = 256
TM = 256
NT = (K * L) // TM + E
RPAD = NT * TM
F_TILE = 2048
NF = F // F_TILE
NSC = 32
A = K * L


def _router_body(x_ref, rn_g_ref, rn_b_ref, wr_ref, br_ref,
                 xn_ref, i1_ref, i2_ref, w1_ref, w2_ref):
    x = x_ref[...]
    mu = jnp.mean(x, axis=-1, keepdims=True)
    xc = x - mu
    var = jnp.mean(xc * xc, axis=-1, keepdims=True)
    xn = xc * lax.rsqrt(var + 1e-5)
    xn_ref[...] = xn
    xa = xn * rn_g_ref[...] + rn_b_ref[...]
    logits = lax.dot_general(
        xa, wr_ref[...], (((1,), (0,)), ((), ())),
        preferred_element_type=jnp.float32) + br_ref[...]
    lane = lax.broadcasted_iota(jnp.int32, logits.shape, 1)
    m1 = jnp.max(logits, axis=-1, keepdims=True)
    i1 = jnp.min(jnp.where(logits == m1, lane, E), axis=-1, keepdims=True)
    l2 = jnp.where(lane == i1, -jnp.inf, logits)
    m2 = jnp.max(l2, axis=-1, keepdims=True)
    i2 = jnp.min(jnp.where(l2 == m2, lane, E), axis=-1, keepdims=True)
    w1 = 1.0 / (1.0 + jnp.exp(m2 - m1))
    w2 = 1.0 - w1
    i1_ref[...] = i1
    i2_ref[...] = i2
    ones = jnp.ones((1, 128), jnp.float32)
    w1_ref[...] = w1 * ones
    w2_ref[...] = w2 * ones


def _router(flat, rn_g, rn_b, Wr, br):
    return pl.pallas_call(
        _router_body,
        grid=(L // ROW_TILE,),
        in_specs=[
            pl.BlockSpec((ROW_TILE, D), lambda i: (i, 0)),
            pl.BlockSpec((1, D), lambda i: (0, 0)),
            pl.BlockSpec((1, D), lambda i: (0, 0)),
            pl.BlockSpec((D, E), lambda i: (0, 0)),
            pl.BlockSpec((1, E), lambda i: (0, 0)),
        ],
        out_specs=[
            pl.BlockSpec((ROW_TILE, D), lambda i: (i, 0)),
            pl.BlockSpec((ROW_TILE, 1), lambda i: (i, 0)),
            pl.BlockSpec((ROW_TILE, 1), lambda i: (i, 0)),
            pl.BlockSpec((ROW_TILE, 128), lambda i: (i, 0)),
            pl.BlockSpec((ROW_TILE, 128), lambda i: (i, 0)),
        ],
        out_shape=[
            jax.ShapeDtypeStruct((L, D), jnp.float32),
            jax.ShapeDtypeStruct((L, 1), jnp.int32),
            jax.ShapeDtypeStruct((L, 1), jnp.int32),
            jax.ShapeDtypeStruct((L, 128), jnp.float32),
            jax.ShapeDtypeStruct((L, 128), jnp.float32),
        ],
    )(flat, rn_g.reshape(1, D), rn_b.reshape(1, D), Wr, br.reshape(1, E))


def _meta_body(e1_ref, e2_ref, pos_ref, te_ref):
    eid = jnp.concatenate([e1_ref[...], e2_ref[...]], axis=0)
    r_iota = lax.broadcasted_iota(jnp.int32, (128, 128), 0)
    c_iota = lax.broadcasted_iota(jnp.int32, (128, 128), 1)
    upper = (r_iota < c_iota).astype(jnp.float32)
    r2 = lax.broadcasted_iota(jnp.int32, (32, 32), 0)
    c2 = lax.broadcasted_iota(jnp.int32, (32, 32), 1)
    lower = (c2 < r2).astype(jnp.float32)

    rank = jnp.zeros((32, 128), jnp.float32)
    offsel = jnp.zeros((32, 128), jnp.float32)
    run = jnp.float32(0.0)
    ends = []
    for e in range(E):
        m = (eid == e).astype(jnp.float32)
        pre = lax.dot_general(m, upper, (((1,), (0,)), ((), ())),
                              preferred_element_type=jnp.float32)
        s = jnp.sum(m, axis=1, keepdims=True)
        roff = lax.dot_general(lower, s, (((1,), (0,)), ((), ())),
                               preferred_element_type=jnp.float32)
        rank = rank + m * (pre + roff)
        offsel = offsel + m * run
        c = jnp.sum(m)
        cpad = jnp.ceil(c / TM) * TM
        run = run + cpad
        ends.append(run)
    pos_ref[...] = (rank + offsel).astype(jnp.int32)

    iv = (lax.broadcasted_iota(jnp.int32, (1, NT), 1) * TM).astype(
        jnp.float32)
    te = jnp.zeros((1, NT), jnp.float32)
    for e in range(E - 1):
        te = te + (iv >= ends[e]).astype(jnp.float32)
    te_ref[...] = te.astype(jnp.int32)


def _metadata(e1r, e2r):
    return pl.pallas_call(
        _meta_body,
        out_shape=[
            jax.ShapeDtypeStruct((32, 128), jnp.int32),
            jax.ShapeDtypeStruct((1, NT), jnp.int32),
        ],
    )(e1r, e2r)


def _sc_scatter_body(xn_hbm, pos_hbm, wb_hbm, xg_hbm, wg_hbm,
                     idx_v, rows_v, wrow_v, sem):
    wid = lax.axis_index("s") * 2 + lax.axis_index("c")
    pltpu.sync_copy(pos_hbm.at[wid], idx_v)
    src0 = lax.rem(wid, 16) * 128
    for j in range(4):
        pltpu.sync_copy(xn_hbm.at[pl.ds(src0 + j * 32, 32)], rows_v)
        pltpu.async_copy(rows_v, xg_hbm.at[idx_v.at[j]], sem).wait()
        pltpu.sync_copy(wb_hbm.at[wid, j], wrow_v)
        pltpu.async_copy(wrow_v, wg_hbm.at[idx_v.at[j]], sem).wait()


def _sc_scatter(xn, pos3, wb):
    mesh = plsc.VectorSubcoreMesh(core_axis_name="c", subcore_axis_name="s")
    fn = functools.partial(
        pl.kernel,
        mesh=mesh,
        out_type=[
            jax.ShapeDtypeStruct((RPAD, D), jnp.float32),
            jax.ShapeDtypeStruct((RPAD, 128), jnp.float32),
        ],
        scratch_types=[
            pltpu.VMEM((4, 32), jnp.int32),
            pltpu.VMEM((32, D), jnp.float32),
            pltpu.VMEM((32, 128), jnp.float32),
            pltpu.SemaphoreType.DMA,
        ],
    )(_sc_scatter_body)
    return fn(xn, pos3, wb)


def _sc_combine_body(og_hbm, p1_hbm, p2_hbm, p1h_hbm, p2h_hbm, out_hbm,
                     idx_v, acc_v, tmp_v, sem):
    wid = lax.axis_index("s") * 2 + lax.axis_index("c")
    pltpu.sync_copy(p1_hbm.at[wid], idx_v.at[0])
    pltpu.sync_copy(p2_hbm.at[wid], idx_v.at[1])
    pltpu.sync_copy(p1h_hbm.at[wid], idx_v.at[2])
    pltpu.sync_copy(p2h_hbm.at[wid], idx_v.at[3])

    def accum(cc, carry):
        off = cc * 16
        for i in range(32):
            acc_v[i, pl.ds(off, 16)] = (
                acc_v[i, pl.ds(off, 16)] + tmp_v[i, pl.ds(off, 16)])
        return carry

    for j in range(2):
        pltpu.async_copy(og_hbm.at[idx_v.at[0, j]], acc_v, sem).wait()
        for q in range(1, 4):
            pltpu.async_copy(og_hbm.at[idx_v.at[q, j]], tmp_v, sem).wait()
            lax.fori_loop(0, D // 16, accum, 0)
        pltpu.sync_copy(acc_v, out_hbm.at[pl.ds(wid * 64 + j * 32, 32)])


def _sc_combine(og2, p1g, p2g, p1h, p2h):
    mesh = plsc.VectorSubcoreMesh(core_axis_name="c", subcore_axis_name="s")
    fn = functools.partial(
        pl.kernel,
        mesh=mesh,
        out_type=jax.ShapeDtypeStruct((L, D), jnp.float32),
        scratch_types=[
            pltpu.VMEM((4, 2, 32), jnp.int32),
            pltpu.VMEM((32, D), jnp.float32),
            pltpu.VMEM((32, D), jnp.float32),
            pltpu.SemaphoreType.DMA,
        ],
    )(_sc_combine_body)
    return fn(og2, p1g, p2g, p1h, p2h)


def _gelu(x):
    return 0.5 * x * (1.0 + lax.erf(x * 0.7071067811865476))


def _ffn_body(te_ref, xg_ref, wg_ref, lng_ref, lnb_ref, w1_ref, b1_ref,
              w2_ref, b2_ref, out_ref):
    f = pl.program_id(0)
    xa = xg_ref[...] * lng_ref[0] + lnb_ref[0]
    h = lax.dot_general(
        xa, w1_ref[0], (((1,), (0,)), ((), ())),
        preferred_element_type=jnp.float32) + b1_ref[0]
    h = _gelu(h)
    o = lax.dot_general(
        h, w2_ref[0], (((1,), (0,)), ((), ())),
        preferred_element_type=jnp.float32)
    o = o + jnp.where(f == 0, b2_ref[0], jnp.zeros_like(b2_ref[0]))
    out_ref[0] = wg_ref[...][:, 0:1] * o


def _grouped_ffn(te1d, xg, wg, ln_g, ln_b, W1, b1, W2, b2):
    grid_spec = pltpu.PrefetchScalarGridSpec(
        num_scalar_prefetch=1,
        grid=(NF, NT),
        in_specs=[
            pl.BlockSpec((TM, D), lambda f, j, te: (j, 0)),
            pl.BlockSpec((TM, 128), lambda f, j, te: (j, 0)),
            pl.BlockSpec((1, 1, D), lambda f, j, te: (te[j], 0, 0)),
            pl.BlockSpec((1, 1, D), lambda f, j, te: (te[j], 0, 0)),
            pl.BlockSpec((1, D, F_TILE), lambda f, j, te: (te[j], 0, f)),
            pl.BlockSpec((1, 1, F_TILE), lambda f, j, te: (te[j], 0, f)),
            pl.BlockSpec((1, F_TILE, D), lambda f, j, te: (te[j], f, 0)),
            pl.BlockSpec((1, 1, D), lambda f, j, te: (te[j], 0, 0)),
        ],
        out_specs=pl.BlockSpec((1, TM, D), lambda f, j, te: (f, j, 0)),
    )
    return pl.pallas_call(
        _ffn_body,
        grid_spec=grid_spec,
        out_shape=jax.ShapeDtypeStruct((NF, RPAD, D), jnp.float32),
    )(te1d, xg, wg, ln_g.reshape(E, 1, D), ln_b.reshape(E, 1, D),
      W1, b1.reshape(E, 1, F), W2, b2.reshape(E, 1, D))


def kernel(hidden_states, rn_g, rn_b, Wr, br, ln_g, ln_b, W1, b1, W2, b2):
    flat = hidden_states.reshape(L, D)
    xn, i1, i2, w1b, w2b = _router(flat, rn_g, rn_b, Wr, br)

    pos2d, te = _metadata(i1.reshape(16, 128), i2.reshape(16, 128))
    pos3 = pos2d.reshape(NSC, 4, 32)
    p1g = pos2d[:16].reshape(NSC, 2, 32)
    p2g = pos2d[16:].reshape(NSC, 2, 32)
    wb = jnp.concatenate([w1b, w2b], axis=0).reshape(NSC, 4, 32, 128)

    xg, wg = _sc_scatter(xn, pos3, wb)
    og = _grouped_ffn(te.reshape(NT), xg, wg, ln_g, ln_b, W1, b1, W2, b2)
    final = _sc_combine(og.reshape(NF * RPAD, D), p1g, p2g,
                        p1g + RPAD, p2g + RPAD)
    return final.reshape(B, L, D)
